# baseline (speedup 1.0000x reference)
import functools

import jax
import jax.numpy as jnp
from jax import lax
from jax.experimental import pallas as pl
from jax.experimental.pallas import tpu as pltpu
from jax.experimental.pallas import tpu_sc as plsc

_NUM_CORES = 2
_NUM_SUBCORES = 16
_NUM_WORKERS = _NUM_CORES * _NUM_SUBCORES


@functools.lru_cache(maxsize=None)
def _make_bcast_rows(b: int, p: int, d: int):
    rows_per_w = p // _NUM_WORKERS
    chunk = rows_per_w
    while 2 * chunk * d * 4 >= 512 * 1024:
        chunk //= 2
    n_chunks = rows_per_w // chunk

    mesh = plsc.VectorSubcoreMesh(core_axis_name="c", subcore_axis_name="s")

    @functools.partial(
        pl.kernel,
        out_type=jax.ShapeDtypeStruct((b, p, d), jnp.float32),
        mesh=mesh,
        scratch_types=[
            pltpu.VMEM((chunk, d), jnp.float32),
            pltpu.VMEM((chunk, d), jnp.float32),
            pltpu.SemaphoreType.DMA,
            pltpu.SemaphoreType.DMA,
            pltpu.SemaphoreType.DMA,
            pltpu.SemaphoreType.DMA,
        ],
    )
    def bcast_rows(wpos_hbm, out_hbm, buf0, buf1, rs0, rs1, ws0, ws1):
        wid = lax.axis_index("s") * _NUM_CORES + lax.axis_index("c")
        base = wid * rows_per_w
        bufs, rsems, wsems = (buf0, buf1), (rs0, rs1), (ws0, ws1)

        reads = {}
        writes = {}

        def start_read(i):
            r0 = base + i * chunk
            reads[i] = pltpu.async_copy(
                wpos_hbm.at[pl.ds(r0, chunk)], bufs[i % 2], rsems[i % 2])

        start_read(0)
        for i in range(n_chunks):
            reads[i].wait()
            if i + 1 < n_chunks:
                if i >= 1:
                    for h in writes[i - 1]:
                        h.wait()
                start_read(i + 1)
            r0 = base + i * chunk
            writes[i] = [
                pltpu.async_copy(
                    bufs[i % 2], out_hbm.at[bi, pl.ds(r0, chunk)], wsems[i % 2])
                for bi in range(b)
            ]
        for i in range(max(0, n_chunks - 2), n_chunks):
            for h in writes[i]:
                h.wait()

    return bcast_rows


def kernel(tokens, W_pos):
    b, p = tokens.shape
    d = W_pos.shape[1]
    return _make_bcast_rows(b, p, d)(W_pos)
